# baseline (speedup 1.0000x reference)
import jax
import jax.numpy as jnp
from jax import lax
from jax.experimental import pallas as pl
from jax.experimental.pallas import tpu as pltpu
from jax.experimental.pallas import tpu_sc as plsc

NC = 2
NS = 16
NW = NC * NS

V = 1000000
NB = 16384
NS_TOK = 26
D = 64
LANES = 128

WBLK = 2 * LANES
N_RBLK = (V - LANES) // LANES + 1
N_WBLK = N_RBLK // 2
WBLK_PER_W = (N_WBLK + NW - 1) // NW
TAIL0 = V - WBLK

B = NB * NS_TOK
B_PER_W = B // NW
N_UNITS_W = B_PER_W // LANES
NB_BLK = NB // LANES


def _build_diag(diag_v, iv):
    def fill(j, carry):
        diag_v[j, :] = lax.rem(iv + j, 16)
        return carry

    lax.fori_loop(0, 16, fill, 0, unroll=4)


def _skew_transpose_pack(src_v, dst_v, n_cols, iv):
    n_mc = n_cols // 16
    rot = [lax.rem(iv + j, 16) for j in range(16)]
    rot_half = [lax.shift_right_logical(r, 1) for r in rot]
    rot_par = [lax.shift_left(jnp.bitwise_and(r, 1), 6) for r in rot]

    def blk16(m, carry):
        cvec = iv + (m // n_mc) * 16
        rbase = (m % n_mc) * 16
        vals = [
            plsc.load_gather(src_v, [cvec, rbase + rot[j]])
            for j in range(16)
        ]
        for j in range(16):
            prow = (rbase >> 1) + rot_half[j]
            pcol = cvec + rot_par[j]
            plsc.store_scatter(dst_v, [prow, pcol], vals[j])
        return carry

    lax.fori_loop(0, 4 * n_mc, blk16, 0, unroll=2)


def _skew_transpose_h(src_v, dst_v, h_all, u, iv):
    rot = [lax.rem(iv + j, 16) for j in range(16)]

    def blk16(m, carry):
        rvec = iv + (m // 4) * 16
        hv = h_all[pl.ds(u * LANES + (m // 4) * 16, 16)]
        cbase = (m % 4) * 16
        vals = [
            plsc.load_gather(src_v, [rvec, cbase + rot[j] + hv])
            for j in range(16)
        ]
        for j in range(16):
            plsc.store_scatter(dst_v, [cbase + rot[j], rvec], vals[j])
        return carry

    lax.fori_loop(0, 8 * 4, blk16, 0, unroll=2)


def _widen_body(tableT_hbm, tail_hbm, wide_hbm, slab0, slab1, slab2,
                blk0, blk1, iota_v, ss0, ss1, ss2, sw0, sw1):
    wid = lax.axis_index("s") * NC + lax.axis_index("c")
    iota_v[...] = lax.iota(jnp.int32, 16)
    iv = iota_v[...]
    slabs = (slab0, slab1, slab2)
    blks = (blk0, blk1)
    sss = (ss0, ss1, ss2)
    sws = (sw0, sw1)

    def r0_of(u):
        return pl.multiple_of((wid + u * NW) * WBLK, LANES)

    def slab_copy(u, sb):
        return pltpu.make_async_copy(
            tableT_hbm.at[:, pl.ds(r0_of(u), WBLK)], slabs[sb], sss[sb]
        )

    def wide_copy(u, par):
        p0 = pl.multiple_of((wid + u * NW) * (WBLK // 2), LANES)
        return pltpu.make_async_copy(
            blks[par], wide_hbm.at[pl.ds(p0, WBLK // 2)], sws[par]
        )

    def in_range(u):
        return wid + u * NW < N_WBLK

    for u0 in (0, 1, 2):
        @pl.when(in_range(u0))
        def _():
            slab_copy(u0, u0).start()

    def phase(k6, p6):
        u = k6 * 6 + p6
        sb = p6 % 3
        par = p6 % 2

        @pl.when(in_range(u))
        def _():
            slab_copy(u, sb).wait()

            @pl.when(u >= 2)
            def _():
                wide_copy(u - 2, par).wait()

            _skew_transpose_pack(slabs[sb], blks[par], WBLK, iv)
            wide_copy(u, par).start()

            @pl.when(in_range(u + 3))
            def _():
                slab_copy(u + 3, sb).start()

    def hexa(k6, carry):
        for p6 in range(6):
            phase(k6, p6)
        return carry

    lax.fori_loop(0, (WBLK_PER_W + 5) // 6, hexa, 0)

    @pl.when(in_range(WBLK_PER_W - 2))
    def _():
        wide_copy(WBLK_PER_W - 2, (WBLK_PER_W - 2) % 2).wait()

    @pl.when(in_range(WBLK_PER_W - 1))
    def _():
        wide_copy(WBLK_PER_W - 1, (WBLK_PER_W - 1) % 2).wait()

    @pl.when(wid == 4)
    def _():
        pltpu.sync_copy(tail_hbm, slab0)
        _skew_transpose_pack(slab0, blk0, WBLK, iv)
        pltpu.sync_copy(blk0, wide_hbm.at[pl.ds(TAIL0 // 2, WBLK // 2)])


def _gather_body(idxf_hbm, wide_hbm, out2_hbm, idx_all, p_all, h_all,
                 rows0, rows1, blk0, blk1, iota_v, sg0, sg1, so0, so1):
    wid = lax.axis_index("s") * NC + lax.axis_index("c")
    iota_v[...] = lax.iota(jnp.int32, 16)
    iv = iota_v[...]
    rows = (rows0, rows1)
    blks = (blk0, blk1)
    sgs = (sg0, sg1)
    sos = (so0, so1)
    uu0 = wid * N_UNITS_W

    pltpu.sync_copy(
        idxf_hbm.at[pl.ds(pl.multiple_of(wid * B_PER_W, 8), B_PER_W)],
        idx_all,
    )

    def presplit(m, carry):
        v = idx_all[pl.ds(m * 16, 16)]
        p_all[pl.ds(m * 16, 16)] = lax.shift_right_logical(v, 1)
        h_all[pl.ds(m * 16, 16)] = lax.shift_left(jnp.bitwise_and(v, 1), 6)
        return carry

    lax.fori_loop(0, B_PER_W // 16, presplit, 0, unroll=4)

    def gather_copy(u, par):
        idx_ref = p_all.at[pl.ds(pl.multiple_of(u * LANES, 8), LANES)]
        return pltpu.make_async_copy(
            wide_hbm.at[idx_ref], rows[par], sgs[par]
        )

    def out_copy(u, par):
        s = (uu0 + u) // NB_BLK
        b0 = pl.multiple_of(((uu0 + u) % NB_BLK) * LANES, LANES)
        return pltpu.make_async_copy(
            blks[par], out2_hbm.at[s, :, pl.ds(b0, LANES)], sos[par]
        )

    gather_copy(0, 0).start()
    gather_copy(1, 1).start()

    def phase(k2, par):
        u = k2 * 2 + par
        gather_copy(u, par).wait()

        @pl.when(k2 >= 1)
        def _():
            out_copy(u - 2, par).wait()

        _skew_transpose_h(rows[par], blks[par], h_all, u, iv)
        out_copy(u, par).start()

        @pl.when(k2 < N_UNITS_W // 2 - 1)
        def _():
            gather_copy(u + 2, par).start()

    def pair(k2, carry):
        phase(k2, 0)
        phase(k2, 1)
        return carry

    lax.fori_loop(0, N_UNITS_W // 2, pair, 0)
    out_copy(N_UNITS_W - 2, 0).wait()
    out_copy(N_UNITS_W - 1, 1).wait()


def kernel(indices, embedding_matrix):
    tableT = embedding_matrix.T
    tail = lax.slice(tableT, (0, TAIL0), (D, V))
    idxT_flat = indices.T.astype(jnp.int32).reshape(-1)
    mesh = plsc.VectorSubcoreMesh(core_axis_name="c", subcore_axis_name="s")
    sc_params = pltpu.CompilerParams(
        use_tc_tiling_on_sc=True, needs_layout_passes=False
    )
    widen = pl.kernel(
        _widen_body,
        out_type=jax.ShapeDtypeStruct((V // 2, LANES), jnp.float32),
        mesh=mesh,
        scratch_types=[
            pltpu.VMEM((D, WBLK), jnp.float32),
            pltpu.VMEM((D, WBLK), jnp.float32),
            pltpu.VMEM((D, WBLK), jnp.float32),
            pltpu.VMEM((WBLK // 2, LANES), jnp.float32),
            pltpu.VMEM((WBLK // 2, LANES), jnp.float32),
            pltpu.VMEM((16,), jnp.int32),
            pltpu.SemaphoreType.DMA,
            pltpu.SemaphoreType.DMA,
            pltpu.SemaphoreType.DMA,
            pltpu.SemaphoreType.DMA,
            pltpu.SemaphoreType.DMA,
        ],
        compiler_params=sc_params,
    )
    gather = pl.kernel(
        _gather_body,
        out_type=jax.ShapeDtypeStruct((NS_TOK, D, NB), jnp.float32),
        mesh=mesh,
        scratch_types=[
            pltpu.VMEM((B_PER_W,), jnp.int32),
            pltpu.VMEM((B_PER_W,), jnp.int32),
            pltpu.VMEM((B_PER_W,), jnp.int32),
            pltpu.VMEM((LANES, LANES), jnp.float32),
            pltpu.VMEM((LANES, LANES), jnp.float32),
            pltpu.VMEM((D, LANES), jnp.float32),
            pltpu.VMEM((D, LANES), jnp.float32),
            pltpu.VMEM((16,), jnp.int32),
            pltpu.SemaphoreType.DMA,
            pltpu.SemaphoreType.DMA,
            pltpu.SemaphoreType.DMA,
            pltpu.SemaphoreType.DMA,
        ],
        compiler_params=sc_params,
    )
    wide = widen(tableT, tail)
    out2 = gather(idxT_flat, wide)
    return jnp.transpose(out2, (2, 0, 1))


# speedup vs baseline: 1.0242x; 1.0242x over previous
import jax
import jax.numpy as jnp
from jax import lax
from jax.experimental import pallas as pl
from jax.experimental.pallas import tpu as pltpu
from jax.experimental.pallas import tpu_sc as plsc

NC = 2
NS = 16
NW = NC * NS

V = 1000000
NB = 16384
NS_TOK = 26
D = 64
LANES = 128

WBLK = 2 * LANES
N_RBLK = (V - LANES) // LANES + 1
N_WBLK = N_RBLK // 2
WBLK_PER_W = (N_WBLK + NW - 1) // NW
TAIL0 = V - WBLK

B = NB * NS_TOK
B_PER_W = B // NW
N_UNITS_W = B_PER_W // LANES
NB_BLK = NB // LANES


def _build_diag(diag_v, iv):
    def fill(j, carry):
        diag_v[j, :] = lax.rem(iv + j, 16)
        return carry

    lax.fori_loop(0, 16, fill, 0, unroll=4)


def _skew_transpose_pack(src_v, dst_v, n_cols, iv):
    n_mc = n_cols // 16
    rot = [lax.rem(iv + j, 16) for j in range(16)]
    rot_half = [lax.shift_right_logical(r, 1) for r in rot]
    rot_par = [lax.shift_left(jnp.bitwise_and(r, 1), 6) for r in rot]

    def blk16(m, carry):
        cvec = iv + (m // n_mc) * 16
        rbase = (m % n_mc) * 16
        vals = [
            plsc.load_gather(src_v, [cvec, rbase + rot[j]])
            for j in range(16)
        ]
        for j in range(16):
            prow = (rbase >> 1) + rot_half[j]
            pcol = cvec + rot_par[j]
            plsc.store_scatter(dst_v, [prow, pcol], vals[j])
        return carry

    lax.fori_loop(0, 4 * n_mc, blk16, 0, unroll=2)


def _skew_transpose_h(src_v, dst_v, h_all, u, iv):
    rot = [lax.rem(iv + j, 16) for j in range(16)]

    def blk16(m, carry):
        rvec = iv + (m // 4) * 16
        hv = h_all[pl.ds(u * LANES + (m // 4) * 16, 16)]
        cbase = (m % 4) * 16
        vals = [
            plsc.load_gather(src_v, [rvec, cbase + rot[j] + hv])
            for j in range(16)
        ]
        for j in range(16):
            plsc.store_scatter(dst_v, [cbase + rot[j], rvec], vals[j])
        return carry

    lax.fori_loop(0, 8 * 4, blk16, 0, unroll=2)


def _widen_body(tableT_hbm, tail_hbm, wide_hbm, slab0, slab1, blk0, blk1,
                iota_v, ss0, ss1, sw0, sw1):
    wid = lax.axis_index("s") * NC + lax.axis_index("c")
    iota_v[...] = lax.iota(jnp.int32, 16)
    iv = iota_v[...]
    slabs = (slab0, slab1)
    blks = (blk0, blk1)
    sss = (ss0, ss1)
    sws = (sw0, sw1)

    def r0_of(u):
        return pl.multiple_of((wid + u * NW) * WBLK, LANES)

    def slab_copy(u, par):
        return pltpu.make_async_copy(
            tableT_hbm.at[:, pl.ds(r0_of(u), WBLK)], slabs[par], sss[par]
        )

    def wide_copy(u, par):
        p0 = pl.multiple_of((wid + u * NW) * (WBLK // 2), LANES)
        return pltpu.make_async_copy(
            blks[par], wide_hbm.at[pl.ds(p0, WBLK // 2)], sws[par]
        )

    def in_range(u):
        return wid + u * NW < N_WBLK

    for par in (0, 1):
        @pl.when(in_range(par))
        def _():
            slab_copy(par, par).start()

    def phase(k2, par):
        u = k2 * 2 + par

        @pl.when(in_range(u))
        def _():
            slab_copy(u, par).wait()

            @pl.when(k2 >= 1)
            def _():
                wide_copy(u - 2, par).wait()

            _skew_transpose_pack(slabs[par], blks[par], WBLK, iv)
            wide_copy(u, par).start()

            @pl.when(in_range(u + 2))
            def _():
                slab_copy(u + 2, par).start()

    def pair(k2, carry):
        phase(k2, 0)
        phase(k2, 1)
        return carry

    lax.fori_loop(0, (WBLK_PER_W + 1) // 2, pair, 0)

    @pl.when(in_range(WBLK_PER_W - 2))
    def _():
        wide_copy(WBLK_PER_W - 2, (WBLK_PER_W - 2) % 2).wait()

    @pl.when(in_range(WBLK_PER_W - 1))
    def _():
        wide_copy(WBLK_PER_W - 1, (WBLK_PER_W - 1) % 2).wait()

    @pl.when(wid == 4)
    def _():
        pltpu.sync_copy(tail_hbm, slab0)
        _skew_transpose_pack(slab0, blk0, WBLK, iv)
        pltpu.sync_copy(blk0, wide_hbm.at[pl.ds(TAIL0 // 2, WBLK // 2)])


def _gather_body(idxf_hbm, wide_hbm, out2_hbm, idx_all, p_all, h_all,
                 rows0, rows1, blk0, blk1, iota_v, sg0, sg1, so0, so1):
    wid = lax.axis_index("s") * NC + lax.axis_index("c")
    iota_v[...] = lax.iota(jnp.int32, 16)
    iv = iota_v[...]
    rows = (rows0, rows1)
    blks = (blk0, blk1)
    sgs = (sg0, sg1)
    sos = (so0, so1)
    uu0 = wid * N_UNITS_W

    pltpu.sync_copy(
        idxf_hbm.at[pl.ds(pl.multiple_of(wid * B_PER_W, 8), B_PER_W)],
        idx_all,
    )

    def presplit(m, carry):
        v = idx_all[pl.ds(m * 16, 16)]
        p_all[pl.ds(m * 16, 16)] = lax.shift_right_logical(v, 1)
        h_all[pl.ds(m * 16, 16)] = lax.shift_left(jnp.bitwise_and(v, 1), 6)
        return carry

    lax.fori_loop(0, B_PER_W // 16, presplit, 0, unroll=4)

    def gather_copy(u, par):
        idx_ref = p_all.at[pl.ds(pl.multiple_of(u * LANES, 8), LANES)]
        return pltpu.make_async_copy(
            wide_hbm.at[idx_ref], rows[par], sgs[par]
        )

    def out_copy(u, par):
        s = (uu0 + u) // NB_BLK
        b0 = pl.multiple_of(((uu0 + u) % NB_BLK) * LANES, LANES)
        return pltpu.make_async_copy(
            blks[par], out2_hbm.at[s, :, pl.ds(b0, LANES)], sos[par]
        )

    gather_copy(0, 0).start()
    gather_copy(1, 1).start()

    def phase(k2, par):
        u = k2 * 2 + par
        gather_copy(u, par).wait()

        @pl.when(k2 >= 1)
        def _():
            out_copy(u - 2, par).wait()

        _skew_transpose_h(rows[par], blks[par], h_all, u, iv)
        out_copy(u, par).start()

        @pl.when(k2 < N_UNITS_W // 2 - 1)
        def _():
            gather_copy(u + 2, par).start()

    def pair(k2, carry):
        phase(k2, 0)
        phase(k2, 1)
        return carry

    lax.fori_loop(0, N_UNITS_W // 2, pair, 0)
    out_copy(N_UNITS_W - 2, 0).wait()
    out_copy(N_UNITS_W - 1, 1).wait()


def kernel(indices, embedding_matrix):
    tableT = embedding_matrix.T
    tail = lax.slice(tableT, (0, TAIL0), (D, V))
    idxT_flat = indices.T.astype(jnp.int32).reshape(-1)
    mesh = plsc.VectorSubcoreMesh(core_axis_name="c", subcore_axis_name="s")
    sc_params = pltpu.CompilerParams(
        use_tc_tiling_on_sc=True, needs_layout_passes=False
    )
    widen = pl.kernel(
        _widen_body,
        out_type=jax.ShapeDtypeStruct((V // 2, LANES), jnp.float32),
        mesh=mesh,
        scratch_types=[
            pltpu.VMEM((D, WBLK), jnp.float32),
            pltpu.VMEM((D, WBLK), jnp.float32),
            pltpu.VMEM((WBLK // 2, LANES), jnp.float32),
            pltpu.VMEM((WBLK // 2, LANES), jnp.float32),
            pltpu.VMEM((16,), jnp.int32),
            pltpu.SemaphoreType.DMA,
            pltpu.SemaphoreType.DMA,
            pltpu.SemaphoreType.DMA,
            pltpu.SemaphoreType.DMA,
        ],
        compiler_params=sc_params,
    )
    gather = pl.kernel(
        _gather_body,
        out_type=jax.ShapeDtypeStruct((NS_TOK, D, NB), jnp.float32),
        mesh=mesh,
        scratch_types=[
            pltpu.VMEM((B_PER_W,), jnp.int32),
            pltpu.VMEM((B_PER_W,), jnp.int32),
            pltpu.VMEM((B_PER_W,), jnp.int32),
            pltpu.VMEM((LANES, LANES), jnp.float32),
            pltpu.VMEM((LANES, LANES), jnp.float32),
            pltpu.VMEM((D, LANES), jnp.float32),
            pltpu.VMEM((D, LANES), jnp.float32),
            pltpu.VMEM((16,), jnp.int32),
            pltpu.SemaphoreType.DMA,
            pltpu.SemaphoreType.DMA,
            pltpu.SemaphoreType.DMA,
            pltpu.SemaphoreType.DMA,
        ],
        compiler_params=sc_params,
    )
    wide = widen(tableT, tail)
    out2 = gather(idxT_flat, wide)
    return jnp.transpose(out2, (2, 0, 1))


# speedup vs baseline: 1.0284x; 1.0041x over previous
import jax
import jax.numpy as jnp
from jax import lax
from jax.experimental import pallas as pl
from jax.experimental.pallas import tpu as pltpu
from jax.experimental.pallas import tpu_sc as plsc

NC = 2
NS = 16
NW = NC * NS

V = 1000000
NB = 16384
NS_TOK = 26
D = 64
LANES = 128

WBLK = 2 * LANES
N_RBLK = (V - LANES) // LANES + 1
N_WBLK = N_RBLK // 2
WBLK_PER_W = (N_WBLK + NW - 1) // NW
TAIL0 = V - WBLK

B = NB * NS_TOK
B_PER_W = B // NW
N_UNITS_W = B_PER_W // LANES
NB_BLK = NB // LANES


def _skew_transpose_pack(src_v, dst_v, n_cols, iv):
    n_mc = n_cols // 16
    rot = [lax.rem(iv + j, 16) for j in range(16)]
    rot_half = [lax.shift_right_logical(r, 1) for r in rot]
    rot_par = [lax.shift_left(jnp.bitwise_and(r, 1), 6) for r in rot]

    def blk16(m, carry):
        cvec = iv + (m // n_mc) * 16
        rbase = (m % n_mc) * 16
        vals = [
            plsc.load_gather(src_v, [cvec, rbase + rot[j]])
            for j in range(16)
        ]
        for j in range(16):
            prow = (rbase >> 1) + rot_half[j]
            pcol = cvec + rot_par[j]
            plsc.store_scatter(dst_v, [prow, pcol], vals[j])
        return carry

    lax.fori_loop(0, 4 * n_mc, blk16, 0, unroll=2)


def _skew_transpose_h(src_v, dst_v, h_all, u, iv):
    rot = [lax.rem(iv + j, 16) for j in range(16)]

    def blk16(m, carry):
        rvec = iv + (m // 4) * 16
        hv = h_all[pl.ds(u * LANES + (m // 4) * 16, 16)]
        cbase = (m % 4) * 16
        vals = [
            plsc.load_gather(src_v, [rvec, cbase + rot[j] + hv])
            for j in range(16)
        ]
        for j in range(16):
            plsc.store_scatter(dst_v, [cbase + rot[j], rvec], vals[j])
        return carry

    lax.fori_loop(0, 8 * 4, blk16, 0, unroll=2)


def _widen_body(tableT_hbm, tail_hbm, wide_hbm, slab0, slab1, blk0, blk1,
                iota_v, ss0, ss1, sw0, sw1):
    wid = lax.axis_index("s") * NC + lax.axis_index("c")
    iota_v[...] = lax.iota(jnp.int32, 16)
    iv = iota_v[...]
    slabs = (slab0, slab1)
    blks = (blk0, blk1)
    sss = (ss0, ss1)
    sws = (sw0, sw1)

    def r0_of(u):
        return pl.multiple_of((wid + u * NW) * WBLK, LANES)

    def slab_copy(u, par):
        return pltpu.make_async_copy(
            tableT_hbm.at[:, pl.ds(r0_of(u), WBLK)], slabs[par], sss[par]
        )

    def wide_copy(u, par):
        p0 = pl.multiple_of((wid + u * NW) * (WBLK // 2), LANES)
        return pltpu.make_async_copy(
            blks[par], wide_hbm.at[pl.ds(p0, WBLK // 2)], sws[par]
        )

    def in_range(u):
        return wid + u * NW < N_WBLK

    for par in (0, 1):
        @pl.when(in_range(par))
        def _():
            slab_copy(par, par).start()

    def phase(k2, par):
        u = k2 * 2 + par

        @pl.when(in_range(u))
        def _():
            slab_copy(u, par).wait()

            @pl.when(k2 >= 1)
            def _():
                wide_copy(u - 2, par).wait()

            _skew_transpose_pack(slabs[par], blks[par], WBLK, iv)
            wide_copy(u, par).start()

            @pl.when(in_range(u + 2))
            def _():
                slab_copy(u + 2, par).start()

    def pair(k2, carry):
        phase(k2, 0)
        phase(k2, 1)
        return carry

    lax.fori_loop(0, (WBLK_PER_W + 1) // 2, pair, 0)

    @pl.when(in_range(WBLK_PER_W - 2))
    def _():
        wide_copy(WBLK_PER_W - 2, (WBLK_PER_W - 2) % 2).wait()

    @pl.when(in_range(WBLK_PER_W - 1))
    def _():
        wide_copy(WBLK_PER_W - 1, (WBLK_PER_W - 1) % 2).wait()

    @pl.when(wid == 4)
    def _():
        pltpu.sync_copy(tail_hbm, slab0)
        _skew_transpose_pack(slab0, blk0, WBLK, iv)
        pltpu.sync_copy(blk0, wide_hbm.at[pl.ds(TAIL0 // 2, WBLK // 2)])


def _gather_body(idxf_hbm, wide_hbm, out2_hbm, idx_all, p_all, h_all,
                 rows0, rows1, blk0, blk1, iota_v, sg0, sg1, so0, so1):
    wid = lax.axis_index("s") * NC + lax.axis_index("c")
    iota_v[...] = lax.iota(jnp.int32, 16)
    iv = iota_v[...]
    rows = (rows0, rows1)
    blks = (blk0, blk1)
    sgs = (sg0, sg1)
    sos = (so0, so1)
    uu0 = wid * N_UNITS_W

    pltpu.sync_copy(
        idxf_hbm.at[pl.ds(pl.multiple_of(wid * B_PER_W, 8), B_PER_W)],
        idx_all,
    )

    def presplit(m, carry):
        v = idx_all[pl.ds(m * 16, 16)]
        p_all[pl.ds(m * 16, 16)] = lax.shift_right_logical(v, 1)
        h_all[pl.ds(m * 16, 16)] = lax.shift_left(jnp.bitwise_and(v, 1), 6)
        return carry

    lax.fori_loop(0, B_PER_W // 16, presplit, 0, unroll=4)

    def gather_copy(u, par):
        idx_ref = p_all.at[pl.ds(pl.multiple_of(u * LANES, 8), LANES)]
        return pltpu.make_async_copy(
            wide_hbm.at[idx_ref], rows[par], sgs[par]
        )

    def out_copy(u, par):
        s = (uu0 + u) // NB_BLK
        b0 = pl.multiple_of(((uu0 + u) % NB_BLK) * LANES, LANES)
        return pltpu.make_async_copy(
            blks[par], out2_hbm.at[s, :, pl.ds(b0, LANES)], sos[par]
        )

    gather_copy(0, 0).start()
    gather_copy(1, 1).start()

    def phase(k2, par):
        u = k2 * 2 + par
        gather_copy(u, par).wait()

        @pl.when(k2 >= 1)
        def _():
            out_copy(u - 2, par).wait()

        _skew_transpose_h(rows[par], blks[par], h_all, u, iv)
        out_copy(u, par).start()

        @pl.when(k2 < N_UNITS_W // 2 - 1)
        def _():
            gather_copy(u + 2, par).start()

    def pair(k2, carry):
        phase(k2, 0)
        phase(k2, 1)
        return carry

    lax.fori_loop(0, N_UNITS_W // 2, pair, 0)
    out_copy(N_UNITS_W - 2, 0).wait()
    out_copy(N_UNITS_W - 1, 1).wait()


def kernel(indices, embedding_matrix):
    tableT = embedding_matrix.T
    tail = lax.slice(tableT, (0, TAIL0), (D, V))
    idxT_flat = indices.T.astype(jnp.int32).reshape(-1)
    mesh = plsc.VectorSubcoreMesh(core_axis_name="c", subcore_axis_name="s")
    sc_params = pltpu.CompilerParams(
        use_tc_tiling_on_sc=True, needs_layout_passes=False
    )
    widen = pl.kernel(
        _widen_body,
        out_type=jax.ShapeDtypeStruct((V // 2, LANES), jnp.float32),
        mesh=mesh,
        scratch_types=[
            pltpu.VMEM((D, WBLK), jnp.float32),
            pltpu.VMEM((D, WBLK), jnp.float32),
            pltpu.VMEM((WBLK // 2, LANES), jnp.float32),
            pltpu.VMEM((WBLK // 2, LANES), jnp.float32),
            pltpu.VMEM((16,), jnp.int32),
            pltpu.SemaphoreType.DMA,
            pltpu.SemaphoreType.DMA,
            pltpu.SemaphoreType.DMA,
            pltpu.SemaphoreType.DMA,
        ],
        compiler_params=sc_params,
    )
    gather = pl.kernel(
        _gather_body,
        out_type=jax.ShapeDtypeStruct((NS_TOK, D, NB), jnp.float32),
        mesh=mesh,
        scratch_types=[
            pltpu.VMEM((B_PER_W,), jnp.int32),
            pltpu.VMEM((B_PER_W,), jnp.int32),
            pltpu.VMEM((B_PER_W,), jnp.int32),
            pltpu.VMEM((LANES, LANES), jnp.float32),
            pltpu.VMEM((LANES, LANES), jnp.float32),
            pltpu.VMEM((D, LANES), jnp.float32),
            pltpu.VMEM((D, LANES), jnp.float32),
            pltpu.VMEM((16,), jnp.int32),
            pltpu.SemaphoreType.DMA,
            pltpu.SemaphoreType.DMA,
            pltpu.SemaphoreType.DMA,
            pltpu.SemaphoreType.DMA,
        ],
        compiler_params=sc_params,
    )
    wide = widen(tableT, tail)
    out2 = gather(idxT_flat, wide)
    return jnp.transpose(out2, (2, 0, 1))
